# Initial kernel scaffold; baseline (speedup 1.0000x reference)
import jax
import jax.numpy as jnp
from jax.experimental import pallas as pl


def kernel(text_queries, motion_features, text_features, k):
    raise NotImplementedError("write your pallas kernel here")


# speedup vs baseline: 1.9815x; 1.9815x over previous
import functools

import jax
import jax.numpy as jnp
from jax import lax
from jax.experimental import pallas as pl
from jax.experimental.pallas import tpu as pltpu
from jax.experimental.pallas import tpu_sc as plsc

EPS = 1e-6
K = 10
NEG = float(jnp.finfo(jnp.float32).min)
IMAX = jnp.iinfo(jnp.int32).max


def _topk_body(nb, n_rows, q_ref, m_ref, topi_ref, topv_s, topi_s):
    qb = q_ref.shape[0]
    bn = m_ref.shape[0]
    j = pl.program_id(1)

    @pl.when(j == 0)
    def _init():
        topv_s[...] = jnp.full((qb, 128), NEG, jnp.float32)
        topi_s[...] = jnp.full((qb, 128), IMAX, jnp.int32)

    q = q_ref[...]
    qn = q / jnp.maximum(
        jnp.sqrt(jnp.sum(q * q, axis=1, keepdims=True)), EPS)
    m = m_ref[...]
    mn = m / jnp.maximum(
        jnp.sqrt(jnp.sum(m * m, axis=1, keepdims=True)), EPS)
    s = lax.dot_general(qn, mn, (((1,), (1,)), ((), ())),
                        preferred_element_type=jnp.float32)
    gidx = j * bn + lax.broadcasted_iota(jnp.int32, (qb, bn), 1)
    s = jnp.where(gidx < n_rows, s, NEG)

    sv = jnp.concatenate([s, topv_s[...]], axis=1)
    si = jnp.concatenate([gidx, topi_s[...]], axis=1)
    lane = lax.broadcasted_iota(jnp.int32, (qb, 128), 1)
    nv = jnp.full((qb, 128), NEG, jnp.float32)
    ni = jnp.full((qb, 128), IMAX, jnp.int32)
    for t in range(K):
        mval = jnp.max(sv, axis=1, keepdims=True)
        aidx = jnp.min(jnp.where(sv == mval, si, IMAX), axis=1, keepdims=True)
        nv = jnp.where(lane == t, mval, nv)
        ni = jnp.where(lane == t, aidx, ni)
        sv = jnp.where(si == aidx, NEG, sv)
    topv_s[...] = nv
    topi_s[...] = ni
    topi_ref[...] = ni


def _topk_indices(q, m):
    qn_rows, d = q.shape
    n = m.shape[0]
    qb = 512 if qn_rows % 512 == 0 else qn_rows
    bn = 2048
    nq = qn_rows // qb
    nb = pl.cdiv(n, bn)
    out = pl.pallas_call(
        functools.partial(_topk_body, nb, n),
        grid=(nq, nb),
        in_specs=[
            pl.BlockSpec((qb, d), lambda i, j: (i, 0)),
            pl.BlockSpec((bn, d), lambda i, j: (j, 0)),
        ],
        out_specs=pl.BlockSpec((qb, 128), lambda i, j: (i, 0)),
        out_shape=jax.ShapeDtypeStruct((qn_rows, 128), jnp.int32),
        scratch_shapes=[
            pltpu.VMEM((qb, 128), jnp.float32),
            pltpu.VMEM((qb, 128), jnp.int32),
        ],
        compiler_params=pltpu.CompilerParams(
            dimension_semantics=("arbitrary", "arbitrary")),
    )(q, m)
    return out[:, :K]


def _make_gather(n_idx, d):
    info = plsc.get_sparse_core_info()
    nw = info.num_cores * info.num_subcores
    per_w = n_idx // nw
    chunk = 64
    n_chunks = per_w // chunk
    mesh = plsc.VectorSubcoreMesh(core_axis_name="c", subcore_axis_name="s")

    @functools.partial(
        pl.kernel,
        mesh=mesh,
        compiler_params=pltpu.CompilerParams(use_tc_tiling_on_sc=False),
        out_type=(
            jax.ShapeDtypeStruct((n_idx, d), jnp.float32),
            jax.ShapeDtypeStruct((n_idx, d), jnp.float32),
        ),
        scratch_types=[
            pltpu.VMEM((per_w,), jnp.int32),
            pltpu.VMEM((per_w, d), jnp.float32),
            pltpu.VMEM((per_w, d), jnp.float32),
            pltpu.SemaphoreType.DMA,
        ],
    )
    def gather2(mot_hbm, txt_hbm, idx_hbm, om_hbm, ot_hbm,
                idx_v, rows_m, rows_t, sem):
        wid = lax.axis_index("s") * info.num_cores + lax.axis_index("c")
        base = wid * per_w
        pltpu.sync_copy(idx_hbm.at[pl.ds(base, per_w)], idx_v)
        copies = []
        for c in range(n_chunks):
            sl = pl.ds(c * chunk, chunk)
            copies.append(
                pltpu.async_copy(mot_hbm.at[idx_v.at[sl]], rows_m.at[sl], sem))
            copies.append(
                pltpu.async_copy(txt_hbm.at[idx_v.at[sl]], rows_t.at[sl], sem))
        for cp in copies:
            cp.wait()
        pltpu.sync_copy(rows_m, om_hbm.at[pl.ds(base, per_w)])
        pltpu.sync_copy(rows_t, ot_hbm.at[pl.ds(base, per_w)])

    return gather2


def kernel(text_queries, motion_features, text_features, k):
    b, d = text_queries.shape
    top_idx = _topk_indices(text_queries, motion_features)
    flat_idx = top_idx.reshape(-1) + (jnp.asarray(k, top_idx.dtype) - K)
    sel_m, sel_t = _make_gather(b * K, d)(
        motion_features, text_features, flat_idx)
    re_motion = sel_m.reshape(b, K, 1, d)
    re_text = sel_t.reshape(b, K, 1, d)
    return (re_text, re_motion)


# speedup vs baseline: 2.8596x; 1.4432x over previous
import functools

import jax
import jax.numpy as jnp
from jax import lax
from jax.experimental import pallas as pl
from jax.experimental.pallas import tpu as pltpu
from jax.experimental.pallas import tpu_sc as plsc

EPS = 1e-6
K = 10
NEG = float(jnp.finfo(jnp.float32).min)
IMAX = jnp.iinfo(jnp.int32).max


def _extract10(vs, gs, qb):
    lane = lax.broadcasted_iota(jnp.int32, (qb, 128), 1)
    nv = jnp.full((qb, 128), NEG, jnp.float32)
    ni = jnp.full((qb, 128), IMAX, jnp.int32)
    for t in range(K):
        mval = vs[0]
        for v in vs[1:]:
            mval = jnp.maximum(mval, v)
        mval = jnp.max(mval, axis=1, keepdims=True)
        cand = jnp.where(vs[0] == mval, gs[0], IMAX)
        for v, g in zip(vs[1:], gs[1:]):
            cand = jnp.minimum(cand, jnp.where(v == mval, g, IMAX))
        aidx = jnp.min(cand, axis=1, keepdims=True)
        nv = jnp.where(lane == t, mval, nv)
        ni = jnp.where(lane == t, aidx, ni)
        vs = [jnp.where(g == aidx, NEG, v) for v, g in zip(vs, gs)]
    return nv, ni


def _topk_body(nb, n_rows, q_ref, m_ref, topi_ref, topv_s, topi_s):
    qb = q_ref.shape[0]
    bn = m_ref.shape[0]
    j = pl.program_id(1)

    @pl.when(j == 0)
    def _init():
        topv_s[...] = jnp.full((qb, 128), NEG, jnp.float32)
        topi_s[...] = jnp.full((qb, 128), IMAX, jnp.int32)

    q = q_ref[...]
    qn = q / jnp.maximum(
        jnp.sqrt(jnp.sum(q * q, axis=1, keepdims=True)), EPS)
    m = m_ref[...]
    mn = m / jnp.maximum(
        jnp.sqrt(jnp.sum(m * m, axis=1, keepdims=True)), EPS)
    s = lax.dot_general(qn, mn, (((1,), (1,)), ((), ())),
                        preferred_element_type=jnp.float32)
    gidx = j * bn + lax.broadcasted_iota(jnp.int32, (qb, bn), 1)
    s = jnp.where(gidx < n_rows, s, NEG)

    carry_v = topv_s[...]
    carry_i = topi_s[...]

    nsl = bn // 128
    c1, g1 = s[:, 0:128], gidx[:, 0:128]
    c2 = jnp.full((qb, 128), NEG, jnp.float32)
    c3, c4 = c2, c2
    g2 = jnp.full((qb, 128), IMAX, jnp.int32)
    g3, g4 = g2, g2
    for kk in range(1, nsl):
        v = s[:, kk * 128:(kk + 1) * 128]
        gv = gidx[:, kk * 128:(kk + 1) * 128]
        gt1, gt2 = v > c1, v > c2
        gt3, gt4 = v > c3, v > c4
        c4 = jnp.where(gt3, c3, jnp.where(gt4, v, c4))
        g4 = jnp.where(gt3, g3, jnp.where(gt4, gv, g4))
        c3 = jnp.where(gt2, c2, jnp.where(gt3, v, c3))
        g3 = jnp.where(gt2, g2, jnp.where(gt3, gv, g3))
        c2 = jnp.where(gt1, c1, jnp.where(gt2, v, c2))
        g2 = jnp.where(gt1, g1, jnp.where(gt2, gv, g2))
        c1 = jnp.where(gt1, v, c1)
        g1 = jnp.where(gt1, gv, g1)

    nv, ni = _extract10([c1, c2, c3, c4, carry_v],
                        [g1, g2, g3, g4, carry_i], qb)

    v10 = jnp.max(jnp.where(
        lax.broadcasted_iota(jnp.int32, (qb, 128), 1) == K - 1, nv, NEG),
        axis=1, keepdims=True)
    viol = jnp.any(c4 >= v10)

    @pl.when(jnp.logical_not(viol))
    def _fast():
        topv_s[...] = nv
        topi_s[...] = ni
        topi_ref[...] = ni

    @pl.when(viol)
    def _slow():
        fv, fi = _extract10(
            [s[:, kk * 128:(kk + 1) * 128] for kk in range(nsl)] + [carry_v],
            [gidx[:, kk * 128:(kk + 1) * 128] for kk in range(nsl)] + [carry_i],
            qb)
        topv_s[...] = fv
        topi_s[...] = fi
        topi_ref[...] = fi


def _topk_indices(q, m):
    qn_rows, d = q.shape
    n = m.shape[0]
    qb = 512 if qn_rows % 512 == 0 else qn_rows
    bn = 2048
    nq = qn_rows // qb
    nb = pl.cdiv(n, bn)
    out = pl.pallas_call(
        functools.partial(_topk_body, nb, n),
        grid=(nq, nb),
        in_specs=[
            pl.BlockSpec((qb, d), lambda i, j: (i, 0)),
            pl.BlockSpec((bn, d), lambda i, j: (j, 0)),
        ],
        out_specs=pl.BlockSpec((qb, 128), lambda i, j: (i, 0)),
        out_shape=jax.ShapeDtypeStruct((qn_rows, 128), jnp.int32),
        scratch_shapes=[
            pltpu.VMEM((qb, 128), jnp.float32),
            pltpu.VMEM((qb, 128), jnp.int32),
        ],
        compiler_params=pltpu.CompilerParams(
            dimension_semantics=("arbitrary", "arbitrary")),
    )(q, m)
    return out[:, :K]


def _make_gather(n_idx, d):
    info = plsc.get_sparse_core_info()
    nw = info.num_cores * info.num_subcores
    per_w = n_idx // nw
    chunk = 64
    n_chunks = per_w // chunk
    mesh = plsc.VectorSubcoreMesh(core_axis_name="c", subcore_axis_name="s")

    @functools.partial(
        pl.kernel,
        mesh=mesh,
        compiler_params=pltpu.CompilerParams(use_tc_tiling_on_sc=False),
        out_type=(
            jax.ShapeDtypeStruct((n_idx, d), jnp.float32),
            jax.ShapeDtypeStruct((n_idx, d), jnp.float32),
        ),
        scratch_types=[
            pltpu.VMEM((per_w,), jnp.int32),
            pltpu.VMEM((per_w, d), jnp.float32),
            pltpu.VMEM((per_w, d), jnp.float32),
            pltpu.SemaphoreType.DMA,
        ],
    )
    def gather2(mot_hbm, txt_hbm, idx_hbm, om_hbm, ot_hbm,
                idx_v, rows_m, rows_t, sem):
        wid = lax.axis_index("s") * info.num_cores + lax.axis_index("c")
        base = wid * per_w
        pltpu.sync_copy(idx_hbm.at[pl.ds(base, per_w)], idx_v)
        copies = []
        for c in range(n_chunks):
            sl = pl.ds(c * chunk, chunk)
            copies.append(
                pltpu.async_copy(mot_hbm.at[idx_v.at[sl]], rows_m.at[sl], sem))
            copies.append(
                pltpu.async_copy(txt_hbm.at[idx_v.at[sl]], rows_t.at[sl], sem))
        for cp in copies:
            cp.wait()
        pltpu.sync_copy(rows_m, om_hbm.at[pl.ds(base, per_w)])
        pltpu.sync_copy(rows_t, ot_hbm.at[pl.ds(base, per_w)])

    return gather2


def kernel(text_queries, motion_features, text_features, k):
    b, d = text_queries.shape
    top_idx = _topk_indices(text_queries, motion_features)
    flat_idx = top_idx.reshape(-1) + (jnp.asarray(k, top_idx.dtype) - K)
    sel_m, sel_t = _make_gather(b * K, d)(
        motion_features, text_features, flat_idx)
    re_motion = sel_m.reshape(b, K, 1, d)
    re_text = sel_t.reshape(b, K, 1, d)
    return (re_text, re_motion)


# speedup vs baseline: 2.8623x; 1.0009x over previous
import functools

import jax
import jax.numpy as jnp
from jax import lax
from jax.experimental import pallas as pl
from jax.experimental.pallas import tpu as pltpu
from jax.experimental.pallas import tpu_sc as plsc

EPS = 1e-6
K = 10
NEG = float(jnp.finfo(jnp.float32).min)
IMAX = jnp.iinfo(jnp.int32).max


def _extract10(vs, gs, qb):
    lane = lax.broadcasted_iota(jnp.int32, (qb, 128), 1)
    nv = jnp.full((qb, 128), NEG, jnp.float32)
    ni = jnp.full((qb, 128), IMAX, jnp.int32)
    for t in range(K):
        mval = vs[0]
        for v in vs[1:]:
            mval = jnp.maximum(mval, v)
        mval = jnp.max(mval, axis=1, keepdims=True)
        cand = jnp.where(vs[0] == mval, gs[0], IMAX)
        for v, g in zip(vs[1:], gs[1:]):
            cand = jnp.minimum(cand, jnp.where(v == mval, g, IMAX))
        aidx = jnp.min(cand, axis=1, keepdims=True)
        nv = jnp.where(lane == t, mval, nv)
        ni = jnp.where(lane == t, aidx, ni)
        vs = [jnp.where(g == aidx, NEG, v) for v, g in zip(vs, gs)]
    return nv, ni


def _topk_body(nb, n_rows, q_ref, m_ref, topi_ref, mn_s, topv_s, topi_s):
    qb = q_ref.shape[0]
    bn = m_ref.shape[0]
    j = pl.program_id(0)
    iq = pl.program_id(1)
    row0 = iq * qb

    @pl.when(jnp.logical_and(j == 0, iq == 0))
    def _init():
        topv_s[...] = jnp.full(topv_s.shape, NEG, jnp.float32)
        topi_s[...] = jnp.full(topi_s.shape, IMAX, jnp.int32)

    @pl.when(iq == 0)
    def _norm_m():
        m = m_ref[...]
        mn_s[...] = m / jnp.maximum(
            jnp.sqrt(jnp.sum(m * m, axis=1, keepdims=True)), EPS)

    q = q_ref[...]
    qn = q / jnp.maximum(
        jnp.sqrt(jnp.sum(q * q, axis=1, keepdims=True)), EPS)
    s = lax.dot_general(qn, mn_s[...], (((1,), (1,)), ((), ())),
                        preferred_element_type=jnp.float32)
    gidx = j * bn + lax.broadcasted_iota(jnp.int32, (qb, bn), 1)
    s = jnp.where(gidx < n_rows, s, NEG)

    carry_v = topv_s[pl.ds(row0, qb)]
    carry_i = topi_s[pl.ds(row0, qb)]

    nsl = bn // 128
    c1, g1 = s[:, 0:128], gidx[:, 0:128]
    c2 = jnp.full((qb, 128), NEG, jnp.float32)
    c3, c4 = c2, c2
    g2 = jnp.full((qb, 128), IMAX, jnp.int32)
    g3, g4 = g2, g2
    for kk in range(1, nsl):
        v = s[:, kk * 128:(kk + 1) * 128]
        gv = gidx[:, kk * 128:(kk + 1) * 128]
        gt1, gt2 = v > c1, v > c2
        gt3, gt4 = v > c3, v > c4
        c4 = jnp.where(gt3, c3, jnp.where(gt4, v, c4))
        g4 = jnp.where(gt3, g3, jnp.where(gt4, gv, g4))
        c3 = jnp.where(gt2, c2, jnp.where(gt3, v, c3))
        g3 = jnp.where(gt2, g2, jnp.where(gt3, gv, g3))
        c2 = jnp.where(gt1, c1, jnp.where(gt2, v, c2))
        g2 = jnp.where(gt1, g1, jnp.where(gt2, gv, g2))
        c1 = jnp.where(gt1, v, c1)
        g1 = jnp.where(gt1, gv, g1)

    c4_orig = c4
    lane = lax.broadcasted_iota(jnp.int32, (qb, 128), 1)
    nv = jnp.full((qb, 128), NEG, jnp.float32)
    ni = jnp.full((qb, 128), IMAX, jnp.int32)
    av, ag, cv, ci = c1, g1, carry_v, carry_i
    for t in range(K):
        mval = jnp.max(jnp.maximum(av, cv), axis=1, keepdims=True)
        cand = jnp.minimum(jnp.where(av == mval, ag, IMAX),
                           jnp.where(cv == mval, ci, IMAX))
        aidx = jnp.min(cand, axis=1, keepdims=True)
        nv = jnp.where(lane == t, mval, nv)
        ni = jnp.where(lane == t, aidx, ni)
        won = ag == aidx
        wonc = ci == aidx
        av = jnp.where(won, c2, av)
        ag = jnp.where(won, g2, ag)
        c2 = jnp.where(won, c3, c2)
        g2 = jnp.where(won, g3, g2)
        c3 = jnp.where(won, c4, c3)
        g3 = jnp.where(won, g4, g3)
        c4 = jnp.where(won, NEG, c4)
        g4 = jnp.where(won, IMAX, g4)
        cv = jnp.where(wonc, NEG, cv)
        ci = jnp.where(wonc, IMAX, ci)

    v10 = jnp.max(jnp.where(lane == K - 1, nv, NEG), axis=1, keepdims=True)
    viol = jnp.any(c4_orig >= v10)

    @pl.when(jnp.logical_not(viol))
    def _fast():
        topv_s[pl.ds(row0, qb)] = nv
        topi_s[pl.ds(row0, qb)] = ni
        topi_ref[pl.ds(row0, qb)] = ni

    @pl.when(viol)
    def _slow():
        fv, fi = _extract10(
            [s[:, kk * 128:(kk + 1) * 128] for kk in range(nsl)] + [carry_v],
            [gidx[:, kk * 128:(kk + 1) * 128] for kk in range(nsl)] + [carry_i],
            qb)
        topv_s[pl.ds(row0, qb)] = fv
        topi_s[pl.ds(row0, qb)] = fi
        topi_ref[pl.ds(row0, qb)] = fi


def _topk_indices(q, m):
    qn_rows, d = q.shape
    n = m.shape[0]
    qb = 512 if qn_rows % 512 == 0 else qn_rows
    bn = 2048
    nq = qn_rows // qb
    nb = pl.cdiv(n, bn)
    out = pl.pallas_call(
        functools.partial(_topk_body, nb, n),
        grid=(nb, nq),
        in_specs=[
            pl.BlockSpec((qb, d), lambda j, i: (i, 0)),
            pl.BlockSpec((bn, d), lambda j, i: (j, 0)),
        ],
        out_specs=pl.BlockSpec((qn_rows, 128), lambda j, i: (0, 0)),
        out_shape=jax.ShapeDtypeStruct((qn_rows, 128), jnp.int32),
        scratch_shapes=[
            pltpu.VMEM((bn, d), jnp.float32),
            pltpu.VMEM((qn_rows, 128), jnp.float32),
            pltpu.VMEM((qn_rows, 128), jnp.int32),
        ],
        compiler_params=pltpu.CompilerParams(
            dimension_semantics=("arbitrary", "arbitrary")),
    )(q, m)
    return out[:, :K]


def _make_gather(n_idx, d):
    info = plsc.get_sparse_core_info()
    nw = info.num_cores * info.num_subcores
    per_w = n_idx // nw
    chunk = 64
    n_chunks = per_w // chunk
    mesh = plsc.VectorSubcoreMesh(core_axis_name="c", subcore_axis_name="s")

    @functools.partial(
        pl.kernel,
        mesh=mesh,
        compiler_params=pltpu.CompilerParams(use_tc_tiling_on_sc=False),
        out_type=(
            jax.ShapeDtypeStruct((n_idx, d), jnp.float32),
            jax.ShapeDtypeStruct((n_idx, d), jnp.float32),
        ),
        scratch_types=[
            pltpu.VMEM((per_w,), jnp.int32),
            pltpu.VMEM((per_w, d), jnp.float32),
            pltpu.VMEM((per_w, d), jnp.float32),
            pltpu.SemaphoreType.DMA,
        ],
    )
    def gather2(mot_hbm, txt_hbm, idx_hbm, om_hbm, ot_hbm,
                idx_v, rows_m, rows_t, sem):
        wid = lax.axis_index("s") * info.num_cores + lax.axis_index("c")
        base = wid * per_w
        pltpu.sync_copy(idx_hbm.at[pl.ds(base, per_w)], idx_v)
        copies = []
        for c in range(n_chunks):
            sl = pl.ds(c * chunk, chunk)
            copies.append(
                pltpu.async_copy(mot_hbm.at[idx_v.at[sl]], rows_m.at[sl], sem))
            copies.append(
                pltpu.async_copy(txt_hbm.at[idx_v.at[sl]], rows_t.at[sl], sem))
        for cp in copies:
            cp.wait()
        pltpu.sync_copy(rows_m, om_hbm.at[pl.ds(base, per_w)])
        pltpu.sync_copy(rows_t, ot_hbm.at[pl.ds(base, per_w)])

    return gather2


def kernel(text_queries, motion_features, text_features, k):
    b, d = text_queries.shape
    top_idx = _topk_indices(text_queries, motion_features)
    flat_idx = top_idx.reshape(-1) + (jnp.asarray(k, top_idx.dtype) - K)
    sel_m, sel_t = _make_gather(b * K, d)(
        motion_features, text_features, flat_idx)
    re_motion = sel_m.reshape(b, K, 1, d)
    re_text = sel_t.reshape(b, K, 1, d)
    return (re_text, re_motion)


# speedup vs baseline: 3.6410x; 1.2721x over previous
import functools

import jax
import jax.numpy as jnp
from jax import lax
from jax.experimental import pallas as pl
from jax.experimental.pallas import tpu as pltpu
from jax.experimental.pallas import tpu_sc as plsc

EPS = 1e-6
K = 10
NEG = float(jnp.finfo(jnp.float32).min)
IMAX = jnp.iinfo(jnp.int32).max


def _extract10(vs, gs, qb):
    lane = lax.broadcasted_iota(jnp.int32, (qb, 128), 1)
    nv = jnp.full((qb, 128), NEG, jnp.float32)
    ni = jnp.full((qb, 128), IMAX, jnp.int32)
    for t in range(K):
        mval = vs[0]
        for v in vs[1:]:
            mval = jnp.maximum(mval, v)
        mval = jnp.max(mval, axis=1, keepdims=True)
        cand = jnp.where(vs[0] == mval, gs[0], IMAX)
        for v, g in zip(vs[1:], gs[1:]):
            cand = jnp.minimum(cand, jnp.where(v == mval, g, IMAX))
        aidx = jnp.min(cand, axis=1, keepdims=True)
        nv = jnp.where(lane == t, mval, nv)
        ni = jnp.where(lane == t, aidx, ni)
        vs = [jnp.where(g == aidx, NEG, v) for v, g in zip(vs, gs)]
    return nv, ni


def _topk_body(nb, n_rows, q_ref, m_ref, topi_ref, mn_s, topv_s, topi_s):
    qb = q_ref.shape[0]
    bn = m_ref.shape[0]
    j = pl.program_id(0)
    iq = pl.program_id(1)
    row0 = iq * qb

    @pl.when(jnp.logical_and(j == 0, iq == 0))
    def _init():
        topv_s[...] = jnp.full(topv_s.shape, NEG, jnp.float32)
        topi_s[...] = jnp.full(topi_s.shape, IMAX, jnp.int32)

    @pl.when(iq == 0)
    def _norm_m():
        m = m_ref[...]
        mn_s[...] = m / jnp.maximum(
            jnp.sqrt(jnp.sum(m * m, axis=1, keepdims=True)), EPS)

    q = q_ref[...]
    qn = q / jnp.maximum(
        jnp.sqrt(jnp.sum(q * q, axis=1, keepdims=True)), EPS)
    s = lax.dot_general(qn, mn_s[...], (((1,), (1,)), ((), ())),
                        preferred_element_type=jnp.float32)
    gidx = j * bn + lax.broadcasted_iota(jnp.int32, (qb, bn), 1)
    s = jnp.where(gidx < n_rows, s, NEG)

    carry_v = topv_s[pl.ds(row0, qb)]
    carry_i = topi_s[pl.ds(row0, qb)]

    nsl = bn // 128
    c1, g1 = s[:, 0:128], gidx[:, 0:128]
    c2 = jnp.full((qb, 128), NEG, jnp.float32)
    c3, c4 = c2, c2
    g2 = jnp.full((qb, 128), IMAX, jnp.int32)
    g3, g4 = g2, g2
    for kk in range(1, nsl):
        v = s[:, kk * 128:(kk + 1) * 128]
        gv = gidx[:, kk * 128:(kk + 1) * 128]
        gt1, gt2 = v > c1, v > c2
        gt3, gt4 = v > c3, v > c4
        c4 = jnp.where(gt3, c3, jnp.where(gt4, v, c4))
        g4 = jnp.where(gt3, g3, jnp.where(gt4, gv, g4))
        c3 = jnp.where(gt2, c2, jnp.where(gt3, v, c3))
        g3 = jnp.where(gt2, g2, jnp.where(gt3, gv, g3))
        c2 = jnp.where(gt1, c1, jnp.where(gt2, v, c2))
        g2 = jnp.where(gt1, g1, jnp.where(gt2, gv, g2))
        c1 = jnp.where(gt1, v, c1)
        g1 = jnp.where(gt1, gv, g1)

    c4_orig = c4
    lane = lax.broadcasted_iota(jnp.int32, (qb, 128), 1)
    nv = jnp.full((qb, 128), NEG, jnp.float32)
    ni = jnp.full((qb, 128), IMAX, jnp.int32)
    av, ag, cv, ci = c1, g1, carry_v, carry_i
    for t in range(K):
        mval = jnp.max(jnp.maximum(av, cv), axis=1, keepdims=True)
        cand = jnp.minimum(jnp.where(av == mval, ag, IMAX),
                           jnp.where(cv == mval, ci, IMAX))
        aidx = jnp.min(cand, axis=1, keepdims=True)
        nv = jnp.where(lane == t, mval, nv)
        ni = jnp.where(lane == t, aidx, ni)
        won = ag == aidx
        wonc = ci == aidx
        av = jnp.where(won, c2, av)
        ag = jnp.where(won, g2, ag)
        c2 = jnp.where(won, c3, c2)
        g2 = jnp.where(won, g3, g2)
        c3 = jnp.where(won, c4, c3)
        g3 = jnp.where(won, g4, g3)
        c4 = jnp.where(won, NEG, c4)
        g4 = jnp.where(won, IMAX, g4)
        cv = jnp.where(wonc, NEG, cv)
        ci = jnp.where(wonc, IMAX, ci)

    v10 = jnp.max(jnp.where(lane == K - 1, nv, NEG), axis=1, keepdims=True)
    viol = jnp.any(c4_orig >= v10)

    @pl.when(jnp.logical_not(viol))
    def _fast():
        topv_s[pl.ds(row0, qb)] = nv
        topi_s[pl.ds(row0, qb)] = ni
        topi_ref[pl.ds(row0, qb)] = ni

    @pl.when(viol)
    def _slow():
        fv, fi = _extract10(
            [s[:, kk * 128:(kk + 1) * 128] for kk in range(nsl)] + [carry_v],
            [gidx[:, kk * 128:(kk + 1) * 128] for kk in range(nsl)] + [carry_i],
            qb)
        topv_s[pl.ds(row0, qb)] = fv
        topi_s[pl.ds(row0, qb)] = fi
        topi_ref[pl.ds(row0, qb)] = fi


def _topk_indices(q, m):
    qn_rows, d = q.shape
    n = m.shape[0]
    qb = 512 if qn_rows % 512 == 0 else qn_rows
    bn = 4096
    nq = qn_rows // qb
    nb = pl.cdiv(n, bn)
    out = pl.pallas_call(
        functools.partial(_topk_body, nb, n),
        grid=(nb, nq),
        in_specs=[
            pl.BlockSpec((qb, d), lambda j, i: (i, 0)),
            pl.BlockSpec((bn, d), lambda j, i: (j, 0)),
        ],
        out_specs=pl.BlockSpec((qn_rows, 128), lambda j, i: (0, 0)),
        out_shape=jax.ShapeDtypeStruct((qn_rows, 128), jnp.int32),
        scratch_shapes=[
            pltpu.VMEM((bn, d), jnp.float32),
            pltpu.VMEM((qn_rows, 128), jnp.float32),
            pltpu.VMEM((qn_rows, 128), jnp.int32),
        ],
        compiler_params=pltpu.CompilerParams(
            dimension_semantics=("arbitrary", "arbitrary")),
    )(q, m)
    return out[:, :K]


def _make_gather(n_idx, d):
    info = plsc.get_sparse_core_info()
    nw = info.num_cores * info.num_subcores
    per_w = n_idx // nw
    chunk = 64
    n_chunks = per_w // chunk
    mesh = plsc.VectorSubcoreMesh(core_axis_name="c", subcore_axis_name="s")

    @functools.partial(
        pl.kernel,
        mesh=mesh,
        compiler_params=pltpu.CompilerParams(use_tc_tiling_on_sc=False),
        out_type=(
            jax.ShapeDtypeStruct((n_idx, d), jnp.float32),
            jax.ShapeDtypeStruct((n_idx, d), jnp.float32),
        ),
        scratch_types=[
            pltpu.VMEM((per_w,), jnp.int32),
            pltpu.VMEM((per_w, d), jnp.float32),
            pltpu.VMEM((per_w, d), jnp.float32),
            pltpu.SemaphoreType.DMA,
        ],
    )
    def gather2(mot_hbm, txt_hbm, idx_hbm, om_hbm, ot_hbm,
                idx_v, rows_m, rows_t, sem):
        wid = lax.axis_index("s") * info.num_cores + lax.axis_index("c")
        base = wid * per_w
        pltpu.sync_copy(idx_hbm.at[pl.ds(base, per_w)], idx_v)
        copies = []
        for c in range(n_chunks):
            sl = pl.ds(c * chunk, chunk)
            copies.append(
                pltpu.async_copy(mot_hbm.at[idx_v.at[sl]], rows_m.at[sl], sem))
            copies.append(
                pltpu.async_copy(txt_hbm.at[idx_v.at[sl]], rows_t.at[sl], sem))
        for cp in copies:
            cp.wait()
        pltpu.sync_copy(rows_m, om_hbm.at[pl.ds(base, per_w)])
        pltpu.sync_copy(rows_t, ot_hbm.at[pl.ds(base, per_w)])

    return gather2


def kernel(text_queries, motion_features, text_features, k):
    b, d = text_queries.shape
    top_idx = _topk_indices(text_queries, motion_features)
    flat_idx = top_idx.reshape(-1) + (jnp.asarray(k, top_idx.dtype) - K)
    sel_m, sel_t = _make_gather(b * K, d)(
        motion_features, text_features, flat_idx)
    re_motion = sel_m.reshape(b, K, 1, d)
    re_text = sel_t.reshape(b, K, 1, d)
    return (re_text, re_motion)


# speedup vs baseline: 4.0229x; 1.1049x over previous
import functools

import jax
import jax.numpy as jnp
from jax import lax
from jax.experimental import pallas as pl
from jax.experimental.pallas import tpu as pltpu
from jax.experimental.pallas import tpu_sc as plsc

EPS = 1e-6
K = 10
NEG = float(jnp.finfo(jnp.float32).min)
IMAX = jnp.iinfo(jnp.int32).max


def _extract10(vs, gs, qb):
    lane = lax.broadcasted_iota(jnp.int32, (qb, 128), 1)
    nv = jnp.full((qb, 128), NEG, jnp.float32)
    ni = jnp.full((qb, 128), IMAX, jnp.int32)
    for t in range(K):
        mval = vs[0]
        for v in vs[1:]:
            mval = jnp.maximum(mval, v)
        mval = jnp.max(mval, axis=1, keepdims=True)
        cand = jnp.where(vs[0] == mval, gs[0], IMAX)
        for v, g in zip(vs[1:], gs[1:]):
            cand = jnp.minimum(cand, jnp.where(v == mval, g, IMAX))
        aidx = jnp.min(cand, axis=1, keepdims=True)
        nv = jnp.where(lane == t, mval, nv)
        ni = jnp.where(lane == t, aidx, ni)
        vs = [jnp.where(g == aidx, NEG, v) for v, g in zip(vs, gs)]
    return nv, ni


def _topk_body(nb, n_rows, q_ref, m_ref, topi_ref, mn_s, topv_s, topi_s):
    qb = q_ref.shape[0]
    bn = m_ref.shape[0]
    j = pl.program_id(0)
    iq = pl.program_id(1)
    row0 = iq * qb

    @pl.when(jnp.logical_and(j == 0, iq == 0))
    def _init():
        topv_s[...] = jnp.full(topv_s.shape, NEG, jnp.float32)
        topi_s[...] = jnp.full(topi_s.shape, IMAX, jnp.int32)

    @pl.when(iq == 0)
    def _norm_m():
        m = m_ref[...]
        mn_s[...] = m / jnp.maximum(
            jnp.sqrt(jnp.sum(m * m, axis=1, keepdims=True)), EPS)

    q = q_ref[...]
    qn = q / jnp.maximum(
        jnp.sqrt(jnp.sum(q * q, axis=1, keepdims=True)), EPS)
    s = lax.dot_general(qn, mn_s[...], (((1,), (1,)), ((), ())),
                        preferred_element_type=jnp.float32)
    gidx = j * bn + lax.broadcasted_iota(jnp.int32, (qb, bn), 1)
    s = jnp.where(gidx < n_rows, s, NEG)

    carry_v = topv_s[pl.ds(row0, qb)]
    carry_i = topi_s[pl.ds(row0, qb)]

    nsl = bn // 128
    c1, g1 = s[:, 0:128], gidx[:, 0:128]
    c2 = jnp.full((qb, 128), NEG, jnp.float32)
    c3, c4 = c2, c2
    g2 = jnp.full((qb, 128), IMAX, jnp.int32)
    g3, g4 = g2, g2
    for kk in range(1, nsl):
        v = s[:, kk * 128:(kk + 1) * 128]
        gv = gidx[:, kk * 128:(kk + 1) * 128]
        gt1, gt2 = v > c1, v > c2
        gt3, gt4 = v > c3, v > c4
        c4 = jnp.where(gt3, c3, jnp.where(gt4, v, c4))
        g4 = jnp.where(gt3, g3, jnp.where(gt4, gv, g4))
        c3 = jnp.where(gt2, c2, jnp.where(gt3, v, c3))
        g3 = jnp.where(gt2, g2, jnp.where(gt3, gv, g3))
        c2 = jnp.where(gt1, c1, jnp.where(gt2, v, c2))
        g2 = jnp.where(gt1, g1, jnp.where(gt2, gv, g2))
        c1 = jnp.where(gt1, v, c1)
        g1 = jnp.where(gt1, gv, g1)

    c4_orig = c4
    lane = lax.broadcasted_iota(jnp.int32, (qb, 128), 1)
    nv = jnp.full((qb, 128), NEG, jnp.float32)
    ni = jnp.full((qb, 128), IMAX, jnp.int32)
    av, ag, cv, ci = c1, g1, carry_v, carry_i
    for t in range(K):
        mval = jnp.max(jnp.maximum(av, cv), axis=1, keepdims=True)
        cand = jnp.minimum(jnp.where(av == mval, ag, IMAX),
                           jnp.where(cv == mval, ci, IMAX))
        aidx = jnp.min(cand, axis=1, keepdims=True)
        nv = jnp.where(lane == t, mval, nv)
        ni = jnp.where(lane == t, aidx, ni)
        won = ag == aidx
        wonc = ci == aidx
        av = jnp.where(won, c2, av)
        ag = jnp.where(won, g2, ag)
        c2 = jnp.where(won, c3, c2)
        g2 = jnp.where(won, g3, g2)
        c3 = jnp.where(won, c4, c3)
        g3 = jnp.where(won, g4, g3)
        c4 = jnp.where(won, NEG, c4)
        g4 = jnp.where(won, IMAX, g4)
        cv = jnp.where(wonc, NEG, cv)
        ci = jnp.where(wonc, IMAX, ci)

    v10 = jnp.max(jnp.where(lane == K - 1, nv, NEG), axis=1, keepdims=True)
    viol = jnp.any(c4_orig >= v10)

    @pl.when(jnp.logical_not(viol))
    def _fast():
        topv_s[pl.ds(row0, qb)] = nv
        topi_s[pl.ds(row0, qb)] = ni
        topi_ref[pl.ds(row0, qb)] = ni

    @pl.when(viol)
    def _slow():
        fv, fi = _extract10(
            [s[:, kk * 128:(kk + 1) * 128] for kk in range(nsl)] + [carry_v],
            [gidx[:, kk * 128:(kk + 1) * 128] for kk in range(nsl)] + [carry_i],
            qb)
        topv_s[pl.ds(row0, qb)] = fv
        topi_s[pl.ds(row0, qb)] = fi
        topi_ref[pl.ds(row0, qb)] = fi


def _topk_indices(q, m):
    qn_rows, d = q.shape
    n = m.shape[0]
    qb = 512 if qn_rows % 512 == 0 else qn_rows
    bn = 8192
    nq = qn_rows // qb
    nb = pl.cdiv(n, bn)
    out = pl.pallas_call(
        functools.partial(_topk_body, nb, n),
        grid=(nb, nq),
        in_specs=[
            pl.BlockSpec((qb, d), lambda j, i: (i, 0)),
            pl.BlockSpec((bn, d), lambda j, i: (j, 0)),
        ],
        out_specs=pl.BlockSpec((qn_rows, 128), lambda j, i: (0, 0)),
        out_shape=jax.ShapeDtypeStruct((qn_rows, 128), jnp.int32),
        scratch_shapes=[
            pltpu.VMEM((bn, d), jnp.float32),
            pltpu.VMEM((qn_rows, 128), jnp.float32),
            pltpu.VMEM((qn_rows, 128), jnp.int32),
        ],
        compiler_params=pltpu.CompilerParams(
            dimension_semantics=("arbitrary", "arbitrary")),
    )(q, m)
    return out[:, :K]


def _make_gather(n_idx, d):
    info = plsc.get_sparse_core_info()
    nw = info.num_cores * info.num_subcores
    per_w = n_idx // nw
    chunk = 64
    n_chunks = per_w // chunk
    mesh = plsc.VectorSubcoreMesh(core_axis_name="c", subcore_axis_name="s")

    @functools.partial(
        pl.kernel,
        mesh=mesh,
        compiler_params=pltpu.CompilerParams(use_tc_tiling_on_sc=False),
        out_type=(
            jax.ShapeDtypeStruct((n_idx, d), jnp.float32),
            jax.ShapeDtypeStruct((n_idx, d), jnp.float32),
        ),
        scratch_types=[
            pltpu.VMEM((per_w,), jnp.int32),
            pltpu.VMEM((per_w, d), jnp.float32),
            pltpu.VMEM((per_w, d), jnp.float32),
            pltpu.SemaphoreType.DMA,
        ],
    )
    def gather2(mot_hbm, txt_hbm, idx_hbm, om_hbm, ot_hbm,
                idx_v, rows_m, rows_t, sem):
        wid = lax.axis_index("s") * info.num_cores + lax.axis_index("c")
        base = wid * per_w
        pltpu.sync_copy(idx_hbm.at[pl.ds(base, per_w)], idx_v)
        copies = []
        for c in range(n_chunks):
            sl = pl.ds(c * chunk, chunk)
            copies.append(
                pltpu.async_copy(mot_hbm.at[idx_v.at[sl]], rows_m.at[sl], sem))
            copies.append(
                pltpu.async_copy(txt_hbm.at[idx_v.at[sl]], rows_t.at[sl], sem))
        for cp in copies:
            cp.wait()
        pltpu.sync_copy(rows_m, om_hbm.at[pl.ds(base, per_w)])
        pltpu.sync_copy(rows_t, ot_hbm.at[pl.ds(base, per_w)])

    return gather2


def kernel(text_queries, motion_features, text_features, k):
    b, d = text_queries.shape
    top_idx = _topk_indices(text_queries, motion_features)
    flat_idx = top_idx.reshape(-1) + (jnp.asarray(k, top_idx.dtype) - K)
    sel_m, sel_t = _make_gather(b * K, d)(
        motion_features, text_features, flat_idx)
    re_motion = sel_m.reshape(b, K, 1, d)
    re_text = sel_t.reshape(b, K, 1, d)
    return (re_text, re_motion)
